# baseline (speedup 1.0000x reference)
import functools

import jax
import jax.numpy as jnp
from jax import lax
from jax.experimental import pallas as pl
from jax.experimental.pallas import tpu as pltpu
from jax.experimental.pallas import tpu_sc as plsc

N_NODES = 10000
N_EDGES = 320000
D_FEAT = 128
D_EDGE = 16
LATENT = 32
OUT_F = 16

_NC = 2
_NS = 16
_NW = _NC * _NS
_EDGES_PER_W = N_EDGES // _NW
_K = 400
_CHUNKS = _EDGES_PER_W // _K

_BE = 12800
_NBLK = N_EDGES // _BE


def _blockdiag(w_ref, reps, bm, bn):
    t = jnp.tile(w_ref[...], (reps, reps))
    ii = lax.broadcasted_iota(jnp.int32, (reps * bm, reps * bn), 0)
    jj = lax.broadcasted_iota(jnp.int32, (reps * bm, reps * bn), 1)
    return jnp.where(ii // bm == jj // bn, t, 0.0)


def _tile4(w, bm, bn):
    t = jnp.tile(w, (4, 4))
    ii = lax.broadcasted_iota(jnp.int32, (4 * bm, 4 * bn), 0)
    jj = lax.broadcasted_iota(jnp.int32, (4 * bm, 4 * bn), 1)
    return jnp.where(ii // bm == jj // bn, t, 0.0)


def _prep_body(node4_ref, w1_ref, g_ref, b1_ref, w2_ref, b2_ref,
               p4_ref, q4_ref, c4_ref, w1e4_ref, w24_ref, b24_ref):
    n4 = node4_ref[...]
    ws = w1_ref[D_EDGE:D_EDGE + D_FEAT, :]
    wr = w1_ref[D_EDGE + D_FEAT:D_EDGE + 2 * D_FEAT, :]
    wg = w1_ref[D_EDGE + 2 * D_FEAT:, :]
    w1e = w1_ref[:D_EDGE, :]
    p4_ref[...] = jnp.dot(n4, _tile4(ws, D_FEAT, LATENT),
                          preferred_element_type=jnp.float32)
    q4_ref[...] = jnp.dot(n4, _tile4(wr, D_FEAT, LATENT),
                          preferred_element_type=jnp.float32)
    c = (jnp.dot(g_ref[...], wg, preferred_element_type=jnp.float32)
         + b1_ref[...])
    c4_ref[...] = jnp.tile(c, (1, 4))
    w1e4_ref[...] = _tile4(w1e, D_EDGE, LATENT)
    w24_ref[...] = _tile4(w2_ref[...], LATENT, OUT_F)
    b24_ref[...] = jnp.tile(b2_ref[...], (1, 4))


def _gather_body(p_hbm, q_hbm, ei_hbm, gs_hbm, gr_hbm,
                 p_sh, q_sh, sidx_v, ridx_v, rp_v, rq_v, gsem, ssem):
    sid = lax.axis_index("s")
    wid = sid * _NC + lax.axis_index("c")
    base = wid * _EDGES_PER_W

    @pl.when(sid == 0)
    def _():
        pltpu.sync_copy(p_hbm, p_sh)
        pltpu.sync_copy(q_hbm, q_sh)

    pltpu.sync_copy(ei_hbm.at[pl.ds(base, _EDGES_PER_W)], sidx_v)
    pltpu.sync_copy(ei_hbm.at[pl.ds(N_EDGES + base, _EDGES_PER_W)], ridx_v)
    plsc.subcore_barrier()

    gw = {}
    sw = {}
    for i in range(_CHUNKS):
        b = i % 2
        if i >= 2:
            sw[i - 2][0].wait()
            sw[i - 2][1].wait()
        gw[i] = (
            pltpu.async_copy(p_sh.at[sidx_v.at[pl.ds(i * _K, _K)]],
                             rp_v[b], gsem[b]),
            pltpu.async_copy(q_sh.at[ridx_v.at[pl.ds(i * _K, _K)]],
                             rq_v[b], gsem[b]),
        )
        if i >= 1:
            pb = (i - 1) % 2
            gw[i - 1][0].wait()
            gw[i - 1][1].wait()
            off = base + (i - 1) * _K
            sw[i - 1] = (
                pltpu.async_copy(rp_v[pb], gs_hbm.at[pl.ds(off, _K)], ssem[pb]),
                pltpu.async_copy(rq_v[pb], gr_hbm.at[pl.ds(off, _K)], ssem[pb]),
            )
    last = _CHUNKS - 1
    lb = last % 2
    gw[last][0].wait()
    gw[last][1].wait()
    off = base + last * _K
    sw[last] = (
        pltpu.async_copy(rp_v[lb], gs_hbm.at[pl.ds(off, _K)], ssem[lb]),
        pltpu.async_copy(rq_v[lb], gr_hbm.at[pl.ds(off, _K)], ssem[lb]),
    )
    sw[last - 1][0].wait()
    sw[last - 1][1].wait()
    sw[last][0].wait()
    sw[last][1].wait()


def _mlp_body(e4in_ref, gs4_ref, gr4_ref, w1e4_ref, c4_ref,
              w24_ref, b24_ref, o_ref):
    epre = (jnp.dot(e4in_ref[...], w1e4_ref[...],
                    preferred_element_type=jnp.float32)
            + c4_ref[...])
    h = epre + gs4_ref[...] + gr4_ref[...]
    h = jnp.maximum(h, 0.0)
    o_ref[...] = (
        jnp.dot(h, w24_ref[...], preferred_element_type=jnp.float32)
        + b24_ref[...]
    )


def kernel(node_attr, edge_index, edge_attr, global_attr, W1, b1, W2, b2):
    b1r = b1.reshape(1, LATENT)
    b2r = b2.reshape(1, OUT_F)
    node4 = node_attr.reshape(N_NODES // 4, 4 * D_FEAT)
    ei_flat = edge_index.astype(jnp.int32).reshape(2 * N_EDGES)

    p4, q4, c4, w1e4, w24, b24 = pl.pallas_call(
        _prep_body,
        out_shape=[
            jax.ShapeDtypeStruct((N_NODES // 4, 4 * LATENT), jnp.float32),
            jax.ShapeDtypeStruct((N_NODES // 4, 4 * LATENT), jnp.float32),
            jax.ShapeDtypeStruct((1, 4 * LATENT), jnp.float32),
            jax.ShapeDtypeStruct((4 * D_EDGE, 4 * LATENT), jnp.float32),
            jax.ShapeDtypeStruct((4 * LATENT, 4 * OUT_F), jnp.float32),
            jax.ShapeDtypeStruct((1, 4 * OUT_F), jnp.float32),
        ],
    )(node4, W1, global_attr, b1r, W2, b2r)

    e4in = edge_attr.reshape(N_EDGES // 4, 4 * D_EDGE)

    sc_gather = pl.kernel(
        _gather_body,
        out_type=[
            jax.ShapeDtypeStruct((N_EDGES, LATENT), jnp.float32),
            jax.ShapeDtypeStruct((N_EDGES, LATENT), jnp.float32),
        ],
        mesh=plsc.VectorSubcoreMesh(core_axis_name="c", subcore_axis_name="s"),
        compiler_params=pltpu.CompilerParams(use_tc_tiling_on_sc=False),
        scratch_types=[
            pltpu.VMEM_SHARED((N_NODES, LATENT), jnp.float32),
            pltpu.VMEM_SHARED((N_NODES, LATENT), jnp.float32),
            pltpu.VMEM((_EDGES_PER_W,), jnp.int32),
            pltpu.VMEM((_EDGES_PER_W,), jnp.int32),
            [pltpu.VMEM((_K, LATENT), jnp.float32) for _ in range(2)],
            [pltpu.VMEM((_K, LATENT), jnp.float32) for _ in range(2)],
            [pltpu.SemaphoreType.DMA for _ in range(2)],
            [pltpu.SemaphoreType.DMA for _ in range(2)],
        ],
    )
    gs, gr = sc_gather(p4.reshape(N_NODES, LATENT),
                       q4.reshape(N_NODES, LATENT),
                       ei_flat)
    gs4 = gs.reshape(N_EDGES // 4, 4 * LATENT)
    gr4 = gr.reshape(N_EDGES // 4, 4 * LATENT)

    ot = pl.pallas_call(
        _mlp_body,
        grid=(_NBLK,),
        in_specs=[
            pl.BlockSpec((_BE // 4, 4 * D_EDGE), lambda i: (i, 0)),
            pl.BlockSpec((_BE // 4, 4 * LATENT), lambda i: (i, 0)),
            pl.BlockSpec((_BE // 4, 4 * LATENT), lambda i: (i, 0)),
            pl.BlockSpec((4 * D_EDGE, 4 * LATENT), lambda i: (0, 0)),
            pl.BlockSpec((1, 4 * LATENT), lambda i: (0, 0)),
            pl.BlockSpec((4 * LATENT, 4 * OUT_F), lambda i: (0, 0)),
            pl.BlockSpec((1, 4 * OUT_F), lambda i: (0, 0)),
        ],
        out_specs=pl.BlockSpec((_BE // 4, 4 * OUT_F), lambda i: (i, 0)),
        out_shape=jax.ShapeDtypeStruct((N_EDGES // 4, 4 * OUT_F),
                                       jnp.float32),
    )(e4in, gs4, gr4, w1e4, c4, w24, b24)

    return ot.reshape(N_EDGES, OUT_F)


# speedup vs baseline: 1.3217x; 1.3217x over previous
import functools

import jax
import jax.numpy as jnp
from jax import lax
from jax.experimental import pallas as pl
from jax.experimental.pallas import tpu as pltpu
from jax.experimental.pallas import tpu_sc as plsc

N_NODES = 10000
N_EDGES = 320000
D_FEAT = 128
D_EDGE = 16
LATENT = 32
OUT_F = 16

_NC = 2
_NS = 16
_NW = _NC * _NS
_EDGES_PER_W = N_EDGES // _NW
_K = 400
_CHUNKS = _EDGES_PER_W // _K

_BE = 12800
_NBLK = N_EDGES // _BE


def _blockdiag(w_ref, reps, bm, bn):
    t = jnp.tile(w_ref[...], (reps, reps))
    ii = lax.broadcasted_iota(jnp.int32, (reps * bm, reps * bn), 0)
    jj = lax.broadcasted_iota(jnp.int32, (reps * bm, reps * bn), 1)
    return jnp.where(ii // bm == jj // bn, t, 0.0)


def _tile4(w, bm, bn):
    t = jnp.tile(w, (4, 4))
    ii = lax.broadcasted_iota(jnp.int32, (4 * bm, 4 * bn), 0)
    jj = lax.broadcasted_iota(jnp.int32, (4 * bm, 4 * bn), 1)
    return jnp.where(ii // bm == jj // bn, t, 0.0)


def _prep_body(node4_ref, w1_ref, g_ref, b1_ref, w2_ref, b2_ref,
               p4_ref, q4_ref, c4_ref, w1e4_ref, w24_ref, b24_ref):
    n4 = node4_ref[...]
    ws = w1_ref[D_EDGE:D_EDGE + D_FEAT, :]
    wr = w1_ref[D_EDGE + D_FEAT:D_EDGE + 2 * D_FEAT, :]
    wg = w1_ref[D_EDGE + 2 * D_FEAT:, :]
    w1e = w1_ref[:D_EDGE, :]
    p4_ref[...] = jnp.dot(n4, _tile4(ws, D_FEAT, LATENT),
                          preferred_element_type=jnp.float32)
    q4_ref[...] = jnp.dot(n4, _tile4(wr, D_FEAT, LATENT),
                          preferred_element_type=jnp.float32)
    c = (jnp.dot(g_ref[...], wg, preferred_element_type=jnp.float32)
         + b1_ref[...])
    c4_ref[...] = jnp.tile(c, (1, 4))
    w1e4_ref[...] = _tile4(w1e, D_EDGE, LATENT)
    w24_ref[...] = _tile4(w2_ref[...], LATENT, OUT_F)
    b24_ref[...] = jnp.tile(b2_ref[...], (1, 4))


def _gather_body(p_hbm, q_hbm, ei_hbm, gs_hbm, gr_hbm,
                 p_sh, q_sh, sidx_v, ridx_v, rp_v, rq_v, gsem, ssem):
    sid = lax.axis_index("s")
    wid = sid * _NC + lax.axis_index("c")
    base = wid * _EDGES_PER_W

    @pl.when(sid == 0)
    def _():
        pltpu.sync_copy(p_hbm, p_sh)
        pltpu.sync_copy(q_hbm, q_sh)

    pltpu.sync_copy(ei_hbm.at[pl.ds(base, _EDGES_PER_W)], sidx_v)
    pltpu.sync_copy(ei_hbm.at[pl.ds(N_EDGES + base, _EDGES_PER_W)], ridx_v)
    plsc.subcore_barrier()

    gw = {}
    sw = {}
    for i in range(_CHUNKS):
        b = i % 2
        if i >= 2:
            sw[i - 2][0].wait()
            sw[i - 2][1].wait()
        gw[i] = (
            pltpu.async_copy(p_sh.at[sidx_v.at[pl.ds(i * _K, _K)]],
                             rp_v[b], gsem[b]),
            pltpu.async_copy(q_sh.at[ridx_v.at[pl.ds(i * _K, _K)]],
                             rq_v[b], gsem[b]),
        )
        if i >= 1:
            pb = (i - 1) % 2
            gw[i - 1][0].wait()
            gw[i - 1][1].wait()
            off = base + (i - 1) * _K
            sw[i - 1] = (
                pltpu.async_copy(rp_v[pb], gs_hbm.at[pl.ds(off, _K)], ssem[pb]),
                pltpu.async_copy(rq_v[pb], gr_hbm.at[pl.ds(off, _K)], ssem[pb]),
            )
    last = _CHUNKS - 1
    lb = last % 2
    gw[last][0].wait()
    gw[last][1].wait()
    off = base + last * _K
    sw[last] = (
        pltpu.async_copy(rp_v[lb], gs_hbm.at[pl.ds(off, _K)], ssem[lb]),
        pltpu.async_copy(rq_v[lb], gr_hbm.at[pl.ds(off, _K)], ssem[lb]),
    )
    sw[last - 1][0].wait()
    sw[last - 1][1].wait()
    sw[last][0].wait()
    sw[last][1].wait()


def _mlp_body(e4in_ref, gs4_ref, gr4_ref, w1e4_ref, c4_ref,
              w24_ref, b24_ref, o_ref):
    epre = (jnp.dot(e4in_ref[...], w1e4_ref[...],
                    preferred_element_type=jnp.float32)
            + c4_ref[...])
    h = epre + gs4_ref[...] + gr4_ref[...]
    h = jnp.maximum(h, 0.0)
    o_ref[...] = (
        jnp.dot(h, w24_ref[...], preferred_element_type=jnp.float32)
        + b24_ref[...]
    )


def kernel(node_attr, edge_index, edge_attr, global_attr, W1, b1, W2, b2):
    b1r = b1.reshape(1, LATENT)
    b2r = b2.reshape(1, OUT_F)
    node4 = node_attr.reshape(N_NODES // 4, 4 * D_FEAT)
    ei_flat = edge_index.astype(jnp.int32).reshape(2 * N_EDGES)

    p4, q4, c4, w1e4, w24, b24 = pl.pallas_call(
        _prep_body,
        out_shape=[
            jax.ShapeDtypeStruct((N_NODES // 4, 4 * LATENT), jnp.float32),
            jax.ShapeDtypeStruct((N_NODES // 4, 4 * LATENT), jnp.float32),
            jax.ShapeDtypeStruct((1, 4 * LATENT), jnp.float32),
            jax.ShapeDtypeStruct((4 * D_EDGE, 4 * LATENT), jnp.float32),
            jax.ShapeDtypeStruct((4 * LATENT, 4 * OUT_F), jnp.float32),
            jax.ShapeDtypeStruct((1, 4 * OUT_F), jnp.float32),
        ],
    )(node4, W1, global_attr, b1r, W2, b2r)

    e4in = jnp.transpose(
        jnp.transpose(edge_attr).reshape(D_EDGE, N_EDGES // 4, 4),
        (1, 2, 0),
    ).reshape(N_EDGES // 4, 4 * D_EDGE)

    sc_gather = pl.kernel(
        _gather_body,
        out_type=[
            jax.ShapeDtypeStruct((N_EDGES, LATENT), jnp.float32),
            jax.ShapeDtypeStruct((N_EDGES, LATENT), jnp.float32),
        ],
        mesh=plsc.VectorSubcoreMesh(core_axis_name="c", subcore_axis_name="s"),
        compiler_params=pltpu.CompilerParams(use_tc_tiling_on_sc=False),
        scratch_types=[
            pltpu.VMEM_SHARED((N_NODES, LATENT), jnp.float32),
            pltpu.VMEM_SHARED((N_NODES, LATENT), jnp.float32),
            pltpu.VMEM((_EDGES_PER_W,), jnp.int32),
            pltpu.VMEM((_EDGES_PER_W,), jnp.int32),
            [pltpu.VMEM((_K, LATENT), jnp.float32) for _ in range(2)],
            [pltpu.VMEM((_K, LATENT), jnp.float32) for _ in range(2)],
            [pltpu.SemaphoreType.DMA for _ in range(2)],
            [pltpu.SemaphoreType.DMA for _ in range(2)],
        ],
    )
    gs, gr = sc_gather(p4.reshape(N_NODES, LATENT),
                       q4.reshape(N_NODES, LATENT),
                       ei_flat)
    gs4 = gs.reshape(N_EDGES // 4, 4 * LATENT)
    gr4 = gr.reshape(N_EDGES // 4, 4 * LATENT)

    ot = pl.pallas_call(
        _mlp_body,
        grid=(_NBLK,),
        in_specs=[
            pl.BlockSpec((_BE // 4, 4 * D_EDGE), lambda i: (i, 0)),
            pl.BlockSpec((_BE // 4, 4 * LATENT), lambda i: (i, 0)),
            pl.BlockSpec((_BE // 4, 4 * LATENT), lambda i: (i, 0)),
            pl.BlockSpec((4 * D_EDGE, 4 * LATENT), lambda i: (0, 0)),
            pl.BlockSpec((1, 4 * LATENT), lambda i: (0, 0)),
            pl.BlockSpec((4 * LATENT, 4 * OUT_F), lambda i: (0, 0)),
            pl.BlockSpec((1, 4 * OUT_F), lambda i: (0, 0)),
        ],
        out_specs=pl.BlockSpec((_BE // 4, 4 * OUT_F), lambda i: (i, 0)),
        out_shape=jax.ShapeDtypeStruct((N_EDGES // 4, 4 * OUT_F),
                                       jnp.float32),
    )(e4in, gs4, gr4, w1e4, c4, w24, b24)

    ott = jnp.transpose(ot.reshape(N_EDGES // 4, 4, OUT_F),
                        (2, 0, 1)).reshape(OUT_F, N_EDGES)
    return jnp.transpose(ott)


# speedup vs baseline: 1.3244x; 1.0021x over previous
import jax
import jax.numpy as jnp
from jax import lax
from jax.experimental import pallas as pl
from jax.experimental.pallas import tpu as pltpu
from jax.experimental.pallas import tpu_sc as plsc

N_NODES = 10000
N_EDGES = 320000
D_FEAT = 128
D_EDGE = 16
LATENT = 32
OUT_F = 16

_NC = 2
_NS = 16
_NW = _NC * _NS
_EDGES_PER_W = N_EDGES // _NW
_K = 400
_CHUNKS = _EDGES_PER_W // _K

_BE = 12800
_NBLK = N_EDGES // _BE


def _tile4(w, bm, bn):
    t = jnp.tile(w, (4, 4))
    ii = lax.broadcasted_iota(jnp.int32, (4 * bm, 4 * bn), 0)
    jj = lax.broadcasted_iota(jnp.int32, (4 * bm, 4 * bn), 1)
    return jnp.where(ii // bm == jj // bn, t, 0.0)


def _prep_body(node4_ref, w1_ref, g_ref, b1_ref, w2_ref, b2_ref,
               p4_ref, q4_ref, c4_ref, w1e4_ref, w24_ref, b24_ref):
    n4 = node4_ref[...]
    ws = w1_ref[D_EDGE:D_EDGE + D_FEAT, :]
    wr = w1_ref[D_EDGE + D_FEAT:D_EDGE + 2 * D_FEAT, :]
    wg = w1_ref[D_EDGE + 2 * D_FEAT:, :]
    w1e = w1_ref[:D_EDGE, :]
    p4_ref[...] = jnp.dot(n4, _tile4(ws, D_FEAT, LATENT),
                          preferred_element_type=jnp.float32)
    q4_ref[...] = jnp.dot(n4, _tile4(wr, D_FEAT, LATENT),
                          preferred_element_type=jnp.float32)
    c = (jnp.dot(g_ref[...], wg, preferred_element_type=jnp.float32)
         + b1_ref[...])
    c4_ref[...] = jnp.tile(c, (1, 4))
    w1e4_ref[...] = _tile4(w1e, D_EDGE, LATENT)
    w24_ref[...] = _tile4(w2_ref[...], LATENT, OUT_F)
    b24_ref[...] = jnp.tile(b2_ref[...], (1, 4))


def _gather_body(p_hbm, q_hbm, ei_hbm, gs_hbm, gr_hbm,
                 p_sh, q_sh, sidx_v, ridx_v, rp_v, rq_v, gsem, ssem):
    sid = lax.axis_index("s")
    wid = sid * _NC + lax.axis_index("c")
    base = wid * _EDGES_PER_W

    @pl.when(sid == 0)
    def _():
        pltpu.sync_copy(p_hbm, p_sh)
        pltpu.sync_copy(q_hbm, q_sh)

    pltpu.sync_copy(ei_hbm.at[pl.ds(base, _EDGES_PER_W)], sidx_v)
    pltpu.sync_copy(ei_hbm.at[pl.ds(N_EDGES + base, _EDGES_PER_W)], ridx_v)
    plsc.subcore_barrier()

    gw = {}
    sw = {}
    for i in range(_CHUNKS):
        b = i % 2
        if i >= 2:
            sw[i - 2][0].wait()
            sw[i - 2][1].wait()
        gw[i] = (
            pltpu.async_copy(p_sh.at[sidx_v.at[pl.ds(i * _K, _K)]],
                             rp_v[b], gsem[b]),
            pltpu.async_copy(q_sh.at[ridx_v.at[pl.ds(i * _K, _K)]],
                             rq_v[b], gsem[b]),
        )
        if i >= 1:
            pb = (i - 1) % 2
            gw[i - 1][0].wait()
            gw[i - 1][1].wait()
            off = base + (i - 1) * _K
            sw[i - 1] = (
                pltpu.async_copy(rp_v[pb], gs_hbm.at[pl.ds(off, _K)], ssem[pb]),
                pltpu.async_copy(rq_v[pb], gr_hbm.at[pl.ds(off, _K)], ssem[pb]),
            )
    last = _CHUNKS - 1
    lb = last % 2
    gw[last][0].wait()
    gw[last][1].wait()
    off = base + last * _K
    sw[last] = (
        pltpu.async_copy(rp_v[lb], gs_hbm.at[pl.ds(off, _K)], ssem[lb]),
        pltpu.async_copy(rq_v[lb], gr_hbm.at[pl.ds(off, _K)], ssem[lb]),
    )
    sw[last - 1][0].wait()
    sw[last - 1][1].wait()
    sw[last][0].wait()
    sw[last][1].wait()


def _mlp_body(e4in_ref, gs4_ref, gr4_ref, w1e4_ref, c4_ref,
              w24_ref, b24_ref, o_ref):
    epre = (jnp.dot(e4in_ref[...], w1e4_ref[...],
                    preferred_element_type=jnp.float32)
            + c4_ref[...])
    h = epre + gs4_ref[...] + gr4_ref[...]
    h = jnp.maximum(h, 0.0)
    o_ref[...] = (
        jnp.dot(h, w24_ref[...], preferred_element_type=jnp.float32)
        + b24_ref[...]
    )


def kernel(node_attr, edge_index, edge_attr, global_attr, W1, b1, W2, b2):
    b1r = b1.reshape(1, LATENT)
    b2r = b2.reshape(1, OUT_F)
    node4 = node_attr.reshape(N_NODES // 4, 4 * D_FEAT)
    ei_flat = edge_index.astype(jnp.int32).reshape(2 * N_EDGES)

    p4, q4, c4, w1e4, w24, b24 = pl.pallas_call(
        _prep_body,
        out_shape=[
            jax.ShapeDtypeStruct((N_NODES // 4, 4 * LATENT), jnp.float32),
            jax.ShapeDtypeStruct((N_NODES // 4, 4 * LATENT), jnp.float32),
            jax.ShapeDtypeStruct((1, 4 * LATENT), jnp.float32),
            jax.ShapeDtypeStruct((4 * D_EDGE, 4 * LATENT), jnp.float32),
            jax.ShapeDtypeStruct((4 * LATENT, 4 * OUT_F), jnp.float32),
            jax.ShapeDtypeStruct((1, 4 * OUT_F), jnp.float32),
        ],
    )(node4, W1, global_attr, b1r, W2, b2r)

    e4in = jnp.transpose(
        jnp.transpose(edge_attr).reshape(D_EDGE, N_EDGES // 4, 4),
        (1, 2, 0),
    ).reshape(N_EDGES // 4, 4 * D_EDGE)

    sc_gather = pl.kernel(
        _gather_body,
        out_type=[
            jax.ShapeDtypeStruct((N_EDGES, LATENT), jnp.float32),
            jax.ShapeDtypeStruct((N_EDGES, LATENT), jnp.float32),
        ],
        mesh=plsc.VectorSubcoreMesh(core_axis_name="c", subcore_axis_name="s"),
        compiler_params=pltpu.CompilerParams(use_tc_tiling_on_sc=False),
        scratch_types=[
            pltpu.VMEM_SHARED((N_NODES, LATENT), jnp.float32),
            pltpu.VMEM_SHARED((N_NODES, LATENT), jnp.float32),
            pltpu.VMEM((_EDGES_PER_W,), jnp.int32),
            pltpu.VMEM((_EDGES_PER_W,), jnp.int32),
            [pltpu.VMEM((_K, LATENT), jnp.float32) for _ in range(2)],
            [pltpu.VMEM((_K, LATENT), jnp.float32) for _ in range(2)],
            [pltpu.SemaphoreType.DMA for _ in range(2)],
            [pltpu.SemaphoreType.DMA for _ in range(2)],
        ],
    )
    gs, gr = sc_gather(p4.reshape(N_NODES, LATENT),
                       q4.reshape(N_NODES, LATENT),
                       ei_flat)
    gs4 = gs.reshape(N_EDGES // 4, 4 * LATENT)
    gr4 = gr.reshape(N_EDGES // 4, 4 * LATENT)

    ot = pl.pallas_call(
        _mlp_body,
        grid=(_NBLK,),
        in_specs=[
            pl.BlockSpec((_BE // 4, 4 * D_EDGE), lambda i: (i, 0)),
            pl.BlockSpec((_BE // 4, 4 * LATENT), lambda i: (i, 0)),
            pl.BlockSpec((_BE // 4, 4 * LATENT), lambda i: (i, 0)),
            pl.BlockSpec((4 * D_EDGE, 4 * LATENT), lambda i: (0, 0)),
            pl.BlockSpec((1, 4 * LATENT), lambda i: (0, 0)),
            pl.BlockSpec((4 * LATENT, 4 * OUT_F), lambda i: (0, 0)),
            pl.BlockSpec((1, 4 * OUT_F), lambda i: (0, 0)),
        ],
        out_specs=pl.BlockSpec((_BE // 4, 4 * OUT_F), lambda i: (i, 0)),
        out_shape=jax.ShapeDtypeStruct((N_EDGES // 4, 4 * OUT_F),
                                       jnp.float32),
    )(e4in, gs4, gr4, w1e4, c4, w24, b24)

    ott = jnp.transpose(ot.reshape(N_EDGES // 4, 4, OUT_F),
                        (2, 0, 1)).reshape(OUT_F, N_EDGES)
    return jnp.transpose(ott)
